# baseline (speedup 1.0000x reference)
import functools

import jax
import jax.numpy as jnp
from jax import lax
from jax.experimental import pallas as pl
from jax.experimental.pallas import tpu as pltpu
from jax.experimental.pallas import tpu_sc as plsc

N = 10000
E = 320000
NSC = 2
NTILES = 16
NP = 10240
RPZ = NP // NTILES
W = 128
SUB = 2000
CHUNK = 80
NCHS = SUB // CHUNK
NBUF = 4


def _sc_mesh():
    return plsc.VectorSubcoreMesh(core_axis_name="c", subcore_axis_name="s")


def _make_agg(splits: int, count: bool):
    ept = (E // (NSC * NTILES)) if splits == 1 else (E // NTILES)
    stages = ept // SUB

    out_type = [jax.ShapeDtypeStruct((NP, W), jnp.float32),
                jax.ShapeDtypeStruct((NP, W), jnp.float32)]
    if count:
        out_type.append(jax.ShapeDtypeStruct((2 * NP,), jnp.float32))

    scratch = [
        pltpu.VMEM((SUB,), jnp.int32),
        pltpu.VMEM((SUB,), jnp.int32),
        pltpu.VMEM_SHARED((NP, W), jnp.float32),
    ]
    for _ in range(NBUF):
        scratch.append(pltpu.VMEM((CHUNK,), jnp.int32))
        scratch.append(pltpu.VMEM((CHUNK,), jnp.int32))
        scratch.append(pltpu.VMEM((CHUNK, W), jnp.float32))
        scratch.append(pltpu.SemaphoreType.DMA)
        scratch.append(pltpu.SemaphoreType.DMA)
    if count:
        scratch.append(pltpu.VMEM((CHUNK,), jnp.float32))
        scratch.append(pltpu.VMEM((RPZ,), jnp.float32))
        scratch.append(pltpu.VMEM_SHARED((NP,), jnp.float32))

    @functools.partial(pl.kernel, out_type=out_type, mesh=_sc_mesh(),
                       scratch_types=scratch,
                       name=f"sage_agg_s{splits}" + ("_cnt" if count else ""))
    def k(*args):
        y, srci, dsti, zrows = args[:4]
        args = args[4:]
        if count:
            zn, ones_in = args[:2]
            args = args[2:]
        outs = args[:2]
        args = args[2:]
        if count:
            cnto = args[0]
            args = args[1:]
        sall, dall, acc = args[:3]
        bufs = [tuple(args[3 + 5 * b:3 + 5 * b + 5]) for b in range(NBUF)]
        if count:
            ones, cbuf, cacc = args[3 + 5 * NBUF:]

        c = lax.axis_index("c")
        s = lax.axis_index("s")
        zsl = pl.ds(s * RPZ, RPZ)

        if count:
            pltpu.sync_copy(ones_in, ones)
            pltpu.sync_copy(zn.at[zsl], cbuf)
            pltpu.sync_copy(cbuf, cacc.at[zsl])

        pltpu.sync_copy(zrows.at[zsl], acc.at[zsl])
        plsc.subcore_barrier()

        def gissue(b, o):
            gix, lix, rws, gsem, _ = bufs[b]
            for i in range(CHUNK // 16):
                st = pl.ds(i * 16, 16)
                dyn = pl.ds(o + i * 16, 16)
                if splits == 1:
                    gix[st] = sall[dyn]
                else:
                    gix[st] = sall[dyn] * 2 + c
                lix[st] = dall[dyn]
            pltpu.async_copy(y.at[gix], rws, gsem)

        def gwait(b):
            gix, _, rws, gsem, _ = bufs[b]
            pltpu.make_async_copy(y.at[gix], rws, gsem).wait()

        def sissue(b):
            _, lix, rws, _, ssem = bufs[b]
            pltpu.async_copy(rws, acc.at[lix], ssem, add=True)
            if count:
                pltpu.sync_copy(ones, cacc.at[lix], add=True)

        def swait(b):
            _, lix, rws, _, ssem = bufs[b]
            pltpu.make_async_copy(rws, acc.at[lix], ssem).wait()

        for h in range(stages):
            if splits == 1:
                ebase = c * (E // 2) + s * ept + h * SUB
            else:
                ebase = s * ept + h * SUB
            pltpu.sync_copy(srci.at[pl.ds(ebase, SUB)], sall)
            pltpu.sync_copy(dsti.at[pl.ds(ebase, SUB)], dall)

            for b in range(NBUF):
                gissue(b, b * CHUNK)

            def body(t, carry):
                o = t * (NBUF * CHUNK)
                for b in range(NBUF):
                    gwait(b)
                    sissue(b)
                for b in range(NBUF):
                    swait(b)
                    gissue(b, o + (NBUF + b) * CHUNK)
                return carry

            lax.fori_loop(0, (NCHS - 1) // NBUF - 1, body, 0)
            for b in range(NBUF):
                gwait(b)
                sissue(b)
            for b in range(NBUF):
                swait(b)
            gissue(0, (NCHS - 1) * CHUNK)
            gwait(0)
            sissue(0)
            swait(0)

        plsc.subcore_barrier()

        @pl.when(c == 0)
        def _():
            pltpu.sync_copy(acc.at[zsl], outs[0].at[zsl])

        @pl.when(c == 1)
        def _():
            pltpu.sync_copy(acc.at[zsl], outs[1].at[zsl])

        if count:
            pltpu.sync_copy(cacc.at[zsl], cbuf)
            pltpu.sync_copy(cbuf, cnto.at[pl.ds(c * NP + s * RPZ, RPZ)])

    return k


_agg_l1 = _make_agg(splits=1, count=True)
_agg_l3 = _make_agg(splits=1, count=False)
_agg_l2 = _make_agg(splits=2, count=False)


_BR = 640


def _row_spec(w):
    return pl.BlockSpec((_BR, w), lambda i: (i, 0))


def _full_spec(a, b):
    return pl.BlockSpec((a, b), lambda i: (0, 0))


def _tc_l1(p0, p1, cnt0, cnt1, x, Wl1, Wr1, b1):
    def body(p0r, p1r, c0r, c1r, xr, wl, wr, br, out, cnto):
        cnt = c0r[...] + c1r[...]
        inv = 1.0 / jnp.maximum(cnt, 1.0)
        z = (jnp.dot((p0r[...] + p1r[...]) * inv, wl[...],
                     preferred_element_type=jnp.float32)
             + jnp.dot(xr[...], wr[...], preferred_element_type=jnp.float32)
             + br[...])
        out[...] = jnp.maximum(z, 0.0)
        cnto[...] = cnt

    return pl.pallas_call(
        body,
        grid=(NP // _BR,),
        in_specs=[_row_spec(128), _row_spec(128), _row_spec(1), _row_spec(1),
                  _row_spec(128), _full_spec(128, 256), _full_spec(128, 256),
                  _full_spec(1, 256)],
        out_specs=[_row_spec(256), _row_spec(1)],
        out_shape=[jax.ShapeDtypeStruct((NP, 256), jnp.float32),
                   jax.ShapeDtypeStruct((NP, 1), jnp.float32)],
    )(p0, p1, cnt0, cnt1, x, Wl1, Wr1, b1)


def _tc_l2(a0, a1, cnt, h1, Wl2a, Wl2b, Wr2, b2, Wl3):
    def body(a0r, a1r, cr, hr, wla, wlb, wr, br, wl3, h2o, y3o):
        inv = 1.0 / jnp.maximum(cr[...], 1.0)
        z = (jnp.dot(a0r[...] * inv, wla[...], preferred_element_type=jnp.float32)
             + jnp.dot(a1r[...] * inv, wlb[...], preferred_element_type=jnp.float32)
             + jnp.dot(hr[...], wr[...], preferred_element_type=jnp.float32)
             + br[...])
        h2 = jnp.maximum(z, 0.0)
        h2o[...] = h2
        y3o[...] = jnp.dot(h2, wl3[...], preferred_element_type=jnp.float32)

    return pl.pallas_call(
        body,
        grid=(NP // _BR,),
        in_specs=[_row_spec(128), _row_spec(128), _row_spec(1),
                  _row_spec(256), _full_spec(128, 256), _full_spec(128, 256),
                  _full_spec(256, 256), _full_spec(1, 256),
                  _full_spec(256, 128)],
        out_specs=[_row_spec(256), _row_spec(128)],
        out_shape=[jax.ShapeDtypeStruct((NP, 256), jnp.float32),
                   jax.ShapeDtypeStruct((NP, 128), jnp.float32)],
    )(a0, a1, cnt, h1, Wl2a, Wl2b, Wr2, b2, Wl3)


def _tc_l3(q0, q1, cnt, h2, Wr3, b3):
    def body(q0r, q1r, cr, hr, wr, br, out):
        inv = 1.0 / jnp.maximum(cr[...], 1.0)
        out[...] = ((q0r[...] + q1r[...]) * inv
                    + jnp.dot(hr[...], wr[...],
                              preferred_element_type=jnp.float32)
                    + br[...])

    return pl.pallas_call(
        body,
        grid=(NP // _BR,),
        in_specs=[_row_spec(128), _row_spec(128), _row_spec(1),
                  _row_spec(256), _full_spec(256, 128), _full_spec(1, 128)],
        out_specs=_row_spec(128),
        out_shape=jax.ShapeDtypeStruct((NP, 128), jnp.float32),
    )(q0, q1, cnt, h2, Wr3, b3)


def kernel(x, edge_index, Wl1, Wr1, b1, Wl2, Wr2, b2, Wl3, Wr3, b3):
    src = edge_index[0].astype(jnp.int32)
    dst = edge_index[1].astype(jnp.int32)
    zrows = jnp.zeros((NP, W), jnp.float32)
    zn = jnp.zeros((NP,), jnp.float32)
    ones = jnp.ones((CHUNK,), jnp.float32)

    xp = jnp.zeros((NP, 128), jnp.float32).at[:N].set(x)
    p0, p1, cnt2 = _agg_l1(xp, src, dst, zrows, zn, ones)
    cnt2 = cnt2.reshape(2, NP, 1)
    h1, cnt = _tc_l1(p0, p1, cnt2[0], cnt2[1], xp, Wl1, Wr1,
                     b1.reshape(1, 256))

    a0, a1 = _agg_l2(h1.reshape(2 * NP, W), src, dst, zrows)
    h2, y3 = _tc_l2(a0, a1, cnt, h1, Wl2[:W], Wl2[W:], Wr2,
                    b2.reshape(1, 256), Wl3)

    q0, q1 = _agg_l3(y3, src, dst, zrows)
    out = _tc_l3(q0, q1, cnt, h2, Wr3, b3.reshape(1, 128))
    return out[:N]


# speedup vs baseline: 1.0833x; 1.0833x over previous
import functools

import jax
import jax.numpy as jnp
from jax import lax
from jax.experimental import pallas as pl
from jax.experimental.pallas import tpu as pltpu
from jax.experimental.pallas import tpu_sc as plsc

N = 10000
E = 320000
NSC = 2
NTILES = 16
NP = 10240
RPZ = NP // NTILES
W = 128
SUB = 10000
CHUNK = 80
NCHS = SUB // CHUNK
NBUF = 2


def _sc_mesh():
    return plsc.VectorSubcoreMesh(core_axis_name="c", subcore_axis_name="s")


def _make_agg(splits: int, count: bool):
    ept = (E // (NSC * NTILES)) if splits == 1 else (E // NTILES)
    stages = ept // SUB

    out_type = [jax.ShapeDtypeStruct((NP, W), jnp.float32),
                jax.ShapeDtypeStruct((NP, W), jnp.float32)]
    if count:
        out_type.append(jax.ShapeDtypeStruct((2 * NP,), jnp.float32))

    scratch = [
        pltpu.VMEM((SUB,), jnp.int32),
        pltpu.VMEM((SUB,), jnp.int32),
        pltpu.VMEM_SHARED((NP, W), jnp.float32),
    ]
    for _ in range(NBUF):
        scratch.append(pltpu.VMEM((CHUNK,), jnp.int32))
        scratch.append(pltpu.VMEM((CHUNK,), jnp.int32))
        scratch.append(pltpu.VMEM((CHUNK, W), jnp.float32))
        scratch.append(pltpu.SemaphoreType.DMA)
        scratch.append(pltpu.SemaphoreType.DMA)
    if count:
        scratch.append(pltpu.VMEM((CHUNK,), jnp.float32))
        scratch.append(pltpu.VMEM((RPZ,), jnp.float32))
        scratch.append(pltpu.VMEM_SHARED((NP,), jnp.float32))

    @functools.partial(pl.kernel, out_type=out_type, mesh=_sc_mesh(),
                       scratch_types=scratch,
                       name=f"sage_agg_s{splits}" + ("_cnt" if count else ""))
    def k(*args):
        y, srci, dsti, zrows = args[:4]
        args = args[4:]
        if count:
            zn, ones_in = args[:2]
            args = args[2:]
        outs = args[:2]
        args = args[2:]
        if count:
            cnto = args[0]
            args = args[1:]
        sall, dall, acc = args[:3]
        bufs = [tuple(args[3 + 5 * b:3 + 5 * b + 5]) for b in range(NBUF)]
        if count:
            ones, cbuf, cacc = args[3 + 5 * NBUF:]

        c = lax.axis_index("c")
        s = lax.axis_index("s")
        zsl = pl.ds(s * RPZ, RPZ)

        if count:
            pltpu.sync_copy(ones_in, ones)
            pltpu.sync_copy(zn.at[zsl], cbuf)
            pltpu.sync_copy(cbuf, cacc.at[zsl])

        pltpu.sync_copy(zrows.at[zsl], acc.at[zsl])
        plsc.subcore_barrier()

        def gissue(b, o):
            gix, lix, rws, gsem, _ = bufs[b]
            for i in range(CHUNK // 16):
                st = pl.ds(i * 16, 16)
                dyn = pl.ds(o + i * 16, 16)
                if splits == 1:
                    gix[st] = sall[dyn]
                else:
                    gix[st] = sall[dyn] * 2 + c
                lix[st] = dall[dyn]
            pltpu.async_copy(y.at[gix], rws, gsem)

        def gwait(b):
            gix, _, rws, gsem, _ = bufs[b]
            pltpu.make_async_copy(y.at[gix], rws, gsem).wait()

        def scat(b):
            _, lix, rws, _, _ = bufs[b]
            pltpu.sync_copy(rws, acc.at[lix], add=True)
            if count:
                pltpu.sync_copy(ones, cacc.at[lix], add=True)

        for h in range(stages):
            if splits == 1:
                ebase = c * (E // 2) + s * ept + h * SUB
            else:
                ebase = s * ept + h * SUB
            pltpu.sync_copy(srci.at[pl.ds(ebase, SUB)], sall)
            pltpu.sync_copy(dsti.at[pl.ds(ebase, SUB)], dall)

            gissue(0, 0)

            def body(t, carry):
                o = t * (2 * CHUNK)
                gissue(1, o + CHUNK)
                gwait(0)
                scat(0)
                gissue(0, o + 2 * CHUNK)
                gwait(1)
                scat(1)
                return carry

            lax.fori_loop(0, (NCHS - 1) // 2, body, 0)
            gwait(0)
            scat(0)

        plsc.subcore_barrier()

        @pl.when(c == 0)
        def _():
            pltpu.sync_copy(acc.at[zsl], outs[0].at[zsl])

        @pl.when(c == 1)
        def _():
            pltpu.sync_copy(acc.at[zsl], outs[1].at[zsl])

        if count:
            pltpu.sync_copy(cacc.at[zsl], cbuf)
            pltpu.sync_copy(cbuf, cnto.at[pl.ds(c * NP + s * RPZ, RPZ)])

    return k


_agg_l1 = _make_agg(splits=1, count=True)
_agg_l3 = _make_agg(splits=1, count=False)
_agg_l2 = _make_agg(splits=2, count=False)


_BR = 640


def _row_spec(w):
    return pl.BlockSpec((_BR, w), lambda i: (i, 0))


def _full_spec(a, b):
    return pl.BlockSpec((a, b), lambda i: (0, 0))


def _tc_l1(p0, p1, cnt0, cnt1, x, Wl1, Wr1, b1):
    def body(p0r, p1r, c0r, c1r, xr, wl, wr, br, out, cnto):
        cnt = c0r[...] + c1r[...]
        inv = 1.0 / jnp.maximum(cnt, 1.0)
        z = (jnp.dot((p0r[...] + p1r[...]) * inv, wl[...],
                     preferred_element_type=jnp.float32)
             + jnp.dot(xr[...], wr[...], preferred_element_type=jnp.float32)
             + br[...])
        out[...] = jnp.maximum(z, 0.0)
        cnto[...] = cnt

    return pl.pallas_call(
        body,
        grid=(NP // _BR,),
        in_specs=[_row_spec(128), _row_spec(128), _row_spec(1), _row_spec(1),
                  _row_spec(128), _full_spec(128, 256), _full_spec(128, 256),
                  _full_spec(1, 256)],
        out_specs=[_row_spec(256), _row_spec(1)],
        out_shape=[jax.ShapeDtypeStruct((NP, 256), jnp.float32),
                   jax.ShapeDtypeStruct((NP, 1), jnp.float32)],
    )(p0, p1, cnt0, cnt1, x, Wl1, Wr1, b1)


def _tc_l2(a0, a1, cnt, h1, Wl2a, Wl2b, Wr2, b2, Wl3):
    def body(a0r, a1r, cr, hr, wla, wlb, wr, br, wl3, h2o, y3o):
        inv = 1.0 / jnp.maximum(cr[...], 1.0)
        z = (jnp.dot(a0r[...] * inv, wla[...], preferred_element_type=jnp.float32)
             + jnp.dot(a1r[...] * inv, wlb[...], preferred_element_type=jnp.float32)
             + jnp.dot(hr[...], wr[...], preferred_element_type=jnp.float32)
             + br[...])
        h2 = jnp.maximum(z, 0.0)
        h2o[...] = h2
        y3o[...] = jnp.dot(h2, wl3[...], preferred_element_type=jnp.float32)

    return pl.pallas_call(
        body,
        grid=(NP // _BR,),
        in_specs=[_row_spec(128), _row_spec(128), _row_spec(1),
                  _row_spec(256), _full_spec(128, 256), _full_spec(128, 256),
                  _full_spec(256, 256), _full_spec(1, 256),
                  _full_spec(256, 128)],
        out_specs=[_row_spec(256), _row_spec(128)],
        out_shape=[jax.ShapeDtypeStruct((NP, 256), jnp.float32),
                   jax.ShapeDtypeStruct((NP, 128), jnp.float32)],
    )(a0, a1, cnt, h1, Wl2a, Wl2b, Wr2, b2, Wl3)


def _tc_l3(q0, q1, cnt, h2, Wr3, b3):
    def body(q0r, q1r, cr, hr, wr, br, out):
        inv = 1.0 / jnp.maximum(cr[...], 1.0)
        out[...] = ((q0r[...] + q1r[...]) * inv
                    + jnp.dot(hr[...], wr[...],
                              preferred_element_type=jnp.float32)
                    + br[...])

    return pl.pallas_call(
        body,
        grid=(NP // _BR,),
        in_specs=[_row_spec(128), _row_spec(128), _row_spec(1),
                  _row_spec(256), _full_spec(256, 128), _full_spec(1, 128)],
        out_specs=_row_spec(128),
        out_shape=jax.ShapeDtypeStruct((NP, 128), jnp.float32),
    )(q0, q1, cnt, h2, Wr3, b3)


def kernel(x, edge_index, Wl1, Wr1, b1, Wl2, Wr2, b2, Wl3, Wr3, b3):
    src = edge_index[0].astype(jnp.int32)
    dst = edge_index[1].astype(jnp.int32)
    zrows = jnp.zeros((NP, W), jnp.float32)
    zn = jnp.zeros((NP,), jnp.float32)
    ones = jnp.ones((CHUNK,), jnp.float32)

    xp = jnp.zeros((NP, 128), jnp.float32).at[:N].set(x)
    p0, p1, cnt2 = _agg_l1(xp, src, dst, zrows, zn, ones)
    cnt2 = cnt2.reshape(2, NP, 1)
    h1, cnt = _tc_l1(p0, p1, cnt2[0], cnt2[1], xp, Wl1, Wr1,
                     b1.reshape(1, 256))

    a0, a1 = _agg_l2(h1.reshape(2 * NP, W), src, dst, zrows)
    h2, y3 = _tc_l2(a0, a1, cnt, h1, Wl2[:W], Wl2[W:], Wr2,
                    b2.reshape(1, 256), Wl3)

    q0, q1 = _agg_l3(y3, src, dst, zrows)
    out = _tc_l3(q0, q1, cnt, h2, Wr3, b3.reshape(1, 128))
    return out[:N]


# speedup vs baseline: 1.1748x; 1.0845x over previous
import functools

import jax
import jax.numpy as jnp
from jax import lax
from jax.experimental import pallas as pl
from jax.experimental.pallas import tpu as pltpu
from jax.experimental.pallas import tpu_sc as plsc

N = 10000
E = 320000
NSC = 2
NTILES = 16
NP = 10240
RPZ = NP // NTILES
W = 128
SUB = 2000
CHUNK = 80
NCHS = SUB // CHUNK
NBUF = 3


def _sc_mesh():
    return plsc.VectorSubcoreMesh(core_axis_name="c", subcore_axis_name="s")


def _make_agg(splits: int, count: bool):
    ept = (E // (NSC * NTILES)) if splits == 1 else (E // NTILES)
    stages = ept // SUB

    out_type = [jax.ShapeDtypeStruct((NP, W), jnp.float32),
                jax.ShapeDtypeStruct((NP, W), jnp.float32)]
    if count:
        out_type.append(jax.ShapeDtypeStruct((2 * NP,), jnp.float32))

    scratch = [
        pltpu.VMEM((SUB,), jnp.int32),
        pltpu.VMEM((SUB,), jnp.int32),
        pltpu.VMEM_SHARED((NP, W), jnp.float32),
    ]
    for _ in range(NBUF):
        scratch.append(pltpu.VMEM((CHUNK,), jnp.int32))
        scratch.append(pltpu.VMEM((CHUNK,), jnp.int32))
        scratch.append(pltpu.VMEM((CHUNK, W), jnp.float32))
        scratch.append(pltpu.SemaphoreType.DMA)
        scratch.append(pltpu.SemaphoreType.DMA)
    if count:
        scratch.append(pltpu.VMEM((CHUNK,), jnp.float32))
        scratch.append(pltpu.VMEM((RPZ,), jnp.float32))
        scratch.append(pltpu.VMEM_SHARED((NP,), jnp.float32))

    @functools.partial(pl.kernel, out_type=out_type, mesh=_sc_mesh(),
                       scratch_types=scratch,
                       name=f"sage_agg_s{splits}" + ("_cnt" if count else ""))
    def k(*args):
        y, srci, dsti, zrows = args[:4]
        args = args[4:]
        if count:
            zn, ones_in = args[:2]
            args = args[2:]
        outs = args[:2]
        args = args[2:]
        if count:
            cnto = args[0]
            args = args[1:]
        sall, dall, acc = args[:3]
        bufs = [tuple(args[3 + 5 * b:3 + 5 * b + 5]) for b in range(NBUF)]
        if count:
            ones, cbuf, cacc = args[3 + 5 * NBUF:]

        c = lax.axis_index("c")
        s = lax.axis_index("s")
        zsl = pl.ds(s * RPZ, RPZ)

        if count:
            pltpu.sync_copy(ones_in, ones)
            pltpu.sync_copy(zn.at[zsl], cbuf)
            pltpu.sync_copy(cbuf, cacc.at[zsl])

        pltpu.sync_copy(zrows.at[zsl], acc.at[zsl])
        plsc.subcore_barrier()

        def gissue(b, o):
            gix, lix, rws, gsem, _ = bufs[b]
            for i in range(CHUNK // 16):
                st = pl.ds(i * 16, 16)
                dyn = pl.ds(o + i * 16, 16)
                if splits == 1:
                    gix[st] = sall[dyn]
                else:
                    gix[st] = sall[dyn] * 2 + c
                lix[st] = dall[dyn]
            pltpu.async_copy(y.at[gix], rws, gsem)

        def gwait(b):
            gix, _, rws, gsem, _ = bufs[b]
            pltpu.make_async_copy(y.at[gix], rws, gsem).wait()

        def scat(b):
            _, lix, rws, _, _ = bufs[b]
            pltpu.sync_copy(rws, acc.at[lix], add=True)
            if count:
                pltpu.sync_copy(ones, cacc.at[lix], add=True)

        for h in range(stages):
            if splits == 1:
                ebase = c * (E // 2) + s * ept + h * SUB
            else:
                ebase = s * ept + h * SUB
            pltpu.sync_copy(srci.at[pl.ds(ebase, SUB)], sall)
            pltpu.sync_copy(dsti.at[pl.ds(ebase, SUB)], dall)

            gissue(0, 0)
            gissue(1, CHUNK)

            def body(t, carry):
                o = t * (3 * CHUNK)
                gissue(2, o + 2 * CHUNK)
                gwait(0)
                scat(0)
                gissue(0, o + 3 * CHUNK)
                gwait(1)
                scat(1)
                gissue(1, o + 4 * CHUNK)
                gwait(2)
                scat(2)
                return carry

            lax.fori_loop(0, (NCHS - 4) // 3, body, 0)
            gissue(2, (NCHS - 2) * CHUNK)
            gwait(0)
            scat(0)
            gissue(0, (NCHS - 1) * CHUNK)
            gwait(1)
            scat(1)
            gwait(2)
            scat(2)
            gwait(0)
            scat(0)

        plsc.subcore_barrier()

        @pl.when(c == 0)
        def _():
            pltpu.sync_copy(acc.at[zsl], outs[0].at[zsl])

        @pl.when(c == 1)
        def _():
            pltpu.sync_copy(acc.at[zsl], outs[1].at[zsl])

        if count:
            pltpu.sync_copy(cacc.at[zsl], cbuf)
            pltpu.sync_copy(cbuf, cnto.at[pl.ds(c * NP + s * RPZ, RPZ)])

    return k


_agg_l1 = _make_agg(splits=1, count=True)
_agg_l3 = _make_agg(splits=1, count=False)
_agg_l2 = _make_agg(splits=2, count=False)


_BR = 640


def _row_spec(w):
    return pl.BlockSpec((_BR, w), lambda i: (i, 0))


def _full_spec(a, b):
    return pl.BlockSpec((a, b), lambda i: (0, 0))


def _tc_l1(p0, p1, cnt0, cnt1, x, Wl1, Wr1, b1):
    def body(p0r, p1r, c0r, c1r, xr, wl, wr, br, out, cnto):
        cnt = c0r[...] + c1r[...]
        inv = 1.0 / jnp.maximum(cnt, 1.0)
        z = (jnp.dot((p0r[...] + p1r[...]) * inv, wl[...],
                     preferred_element_type=jnp.float32)
             + jnp.dot(xr[...], wr[...], preferred_element_type=jnp.float32)
             + br[...])
        out[...] = jnp.maximum(z, 0.0)
        cnto[...] = cnt

    return pl.pallas_call(
        body,
        grid=(NP // _BR,),
        in_specs=[_row_spec(128), _row_spec(128), _row_spec(1), _row_spec(1),
                  _row_spec(128), _full_spec(128, 256), _full_spec(128, 256),
                  _full_spec(1, 256)],
        out_specs=[_row_spec(256), _row_spec(1)],
        out_shape=[jax.ShapeDtypeStruct((NP, 256), jnp.float32),
                   jax.ShapeDtypeStruct((NP, 1), jnp.float32)],
    )(p0, p1, cnt0, cnt1, x, Wl1, Wr1, b1)


def _tc_l2(a0, a1, cnt, h1, Wl2a, Wl2b, Wr2, b2, Wl3):
    def body(a0r, a1r, cr, hr, wla, wlb, wr, br, wl3, h2o, y3o):
        inv = 1.0 / jnp.maximum(cr[...], 1.0)
        z = (jnp.dot(a0r[...] * inv, wla[...], preferred_element_type=jnp.float32)
             + jnp.dot(a1r[...] * inv, wlb[...], preferred_element_type=jnp.float32)
             + jnp.dot(hr[...], wr[...], preferred_element_type=jnp.float32)
             + br[...])
        h2 = jnp.maximum(z, 0.0)
        h2o[...] = h2
        y3o[...] = jnp.dot(h2, wl3[...], preferred_element_type=jnp.float32)

    return pl.pallas_call(
        body,
        grid=(NP // _BR,),
        in_specs=[_row_spec(128), _row_spec(128), _row_spec(1),
                  _row_spec(256), _full_spec(128, 256), _full_spec(128, 256),
                  _full_spec(256, 256), _full_spec(1, 256),
                  _full_spec(256, 128)],
        out_specs=[_row_spec(256), _row_spec(128)],
        out_shape=[jax.ShapeDtypeStruct((NP, 256), jnp.float32),
                   jax.ShapeDtypeStruct((NP, 128), jnp.float32)],
    )(a0, a1, cnt, h1, Wl2a, Wl2b, Wr2, b2, Wl3)


def _tc_l3(q0, q1, cnt, h2, Wr3, b3):
    def body(q0r, q1r, cr, hr, wr, br, out):
        inv = 1.0 / jnp.maximum(cr[...], 1.0)
        out[...] = ((q0r[...] + q1r[...]) * inv
                    + jnp.dot(hr[...], wr[...],
                              preferred_element_type=jnp.float32)
                    + br[...])

    return pl.pallas_call(
        body,
        grid=(NP // _BR,),
        in_specs=[_row_spec(128), _row_spec(128), _row_spec(1),
                  _row_spec(256), _full_spec(256, 128), _full_spec(1, 128)],
        out_specs=_row_spec(128),
        out_shape=jax.ShapeDtypeStruct((NP, 128), jnp.float32),
    )(q0, q1, cnt, h2, Wr3, b3)


def kernel(x, edge_index, Wl1, Wr1, b1, Wl2, Wr2, b2, Wl3, Wr3, b3):
    src = edge_index[0].astype(jnp.int32)
    dst = edge_index[1].astype(jnp.int32)
    zrows = jnp.zeros((NP, W), jnp.float32)
    zn = jnp.zeros((NP,), jnp.float32)
    ones = jnp.ones((CHUNK,), jnp.float32)

    xp = jnp.zeros((NP, 128), jnp.float32).at[:N].set(x)
    p0, p1, cnt2 = _agg_l1(xp, src, dst, zrows, zn, ones)
    cnt2 = cnt2.reshape(2, NP, 1)
    h1, cnt = _tc_l1(p0, p1, cnt2[0], cnt2[1], xp, Wl1, Wr1,
                     b1.reshape(1, 256))

    a0, a1 = _agg_l2(h1.reshape(2 * NP, W), src, dst, zrows)
    h2, y3 = _tc_l2(a0, a1, cnt, h1, Wl2[:W], Wl2[W:], Wr2,
                    b2.reshape(1, 256), Wl3)

    q0, q1 = _agg_l3(y3, src, dst, zrows)
    out = _tc_l3(q0, q1, cnt, h2, Wr3, b3.reshape(1, 128))
    return out[:N]


# speedup vs baseline: 1.1751x; 1.0002x over previous
import functools

import jax
import jax.numpy as jnp
from jax import lax
from jax.experimental import pallas as pl
from jax.experimental.pallas import tpu as pltpu
from jax.experimental.pallas import tpu_sc as plsc

N = 10000
E = 320000
NSC = 2
NTILES = 16
NP = 10240
RPZ = NP // NTILES
W = 128
SUB = 2000
CHUNK = 80
NCHS = SUB // CHUNK
NBUF = 4


def _sc_mesh():
    return plsc.VectorSubcoreMesh(core_axis_name="c", subcore_axis_name="s")


def _make_agg(splits: int, count: bool):
    ept = (E // (NSC * NTILES)) if splits == 1 else (E // NTILES)
    stages = ept // SUB

    out_type = [jax.ShapeDtypeStruct((NP, W), jnp.float32),
                jax.ShapeDtypeStruct((NP, W), jnp.float32)]
    if count:
        out_type.append(jax.ShapeDtypeStruct((2 * NP,), jnp.float32))

    scratch = [
        pltpu.VMEM((SUB,), jnp.int32),
        pltpu.VMEM((SUB,), jnp.int32),
        pltpu.VMEM_SHARED((NP, W), jnp.float32),
    ]
    for _ in range(NBUF):
        scratch.append(pltpu.VMEM((CHUNK,), jnp.int32))
        scratch.append(pltpu.VMEM((CHUNK,), jnp.int32))
        scratch.append(pltpu.VMEM((CHUNK, W), jnp.float32))
        scratch.append(pltpu.SemaphoreType.DMA)
        scratch.append(pltpu.SemaphoreType.DMA)
    if count:
        scratch.append(pltpu.VMEM((CHUNK,), jnp.float32))
        scratch.append(pltpu.VMEM((RPZ,), jnp.float32))
        scratch.append(pltpu.VMEM_SHARED((NP,), jnp.float32))

    @functools.partial(pl.kernel, out_type=out_type, mesh=_sc_mesh(),
                       scratch_types=scratch,
                       name=f"sage_agg_s{splits}" + ("_cnt" if count else ""))
    def k(*args):
        y, srci, dsti, zrows = args[:4]
        args = args[4:]
        if count:
            zn, ones_in = args[:2]
            args = args[2:]
        outs = args[:2]
        args = args[2:]
        if count:
            cnto = args[0]
            args = args[1:]
        sall, dall, acc = args[:3]
        bufs = [tuple(args[3 + 5 * b:3 + 5 * b + 5]) for b in range(NBUF)]
        if count:
            ones, cbuf, cacc = args[3 + 5 * NBUF:]

        c = lax.axis_index("c")
        s = lax.axis_index("s")
        zsl = pl.ds(s * RPZ, RPZ)

        if count:
            pltpu.sync_copy(ones_in, ones)
            pltpu.sync_copy(zn.at[zsl], cbuf)
            pltpu.sync_copy(cbuf, cacc.at[zsl])

        pltpu.sync_copy(zrows.at[zsl], acc.at[zsl])
        plsc.subcore_barrier()

        def gissue(b, o):
            gix, lix, rws, gsem, _ = bufs[b]
            for i in range(CHUNK // 16):
                st = pl.ds(i * 16, 16)
                dyn = pl.ds(o + i * 16, 16)
                if splits == 1:
                    gix[st] = sall[dyn]
                else:
                    gix[st] = sall[dyn] * 2 + c
                lix[st] = dall[dyn]
            pltpu.async_copy(y.at[gix], rws, gsem)

        def gwait(b):
            gix, _, rws, gsem, _ = bufs[b]
            pltpu.make_async_copy(y.at[gix], rws, gsem).wait()

        def scat(b):
            _, lix, rws, _, _ = bufs[b]
            pltpu.sync_copy(rws, acc.at[lix], add=True)
            if count:
                pltpu.sync_copy(ones, cacc.at[lix], add=True)

        for h in range(stages):
            if splits == 1:
                ebase = c * (E // 2) + s * ept + h * SUB
            else:
                ebase = s * ept + h * SUB
            pltpu.sync_copy(srci.at[pl.ds(ebase, SUB)], sall)
            pltpu.sync_copy(dsti.at[pl.ds(ebase, SUB)], dall)

            gissue(0, 0)
            gissue(1, CHUNK)
            gissue(2, 2 * CHUNK)

            def body(t, carry):
                o = t * (4 * CHUNK)
                gissue(3, o + 3 * CHUNK)
                gwait(0)
                scat(0)
                gissue(0, o + 4 * CHUNK)
                gwait(1)
                scat(1)
                gissue(1, o + 5 * CHUNK)
                gwait(2)
                scat(2)
                gissue(2, o + 6 * CHUNK)
                gwait(3)
                scat(3)
                return carry

            lax.fori_loop(0, (NCHS - 5) // 4, body, 0)
            gissue(3, (NCHS - 2) * CHUNK)
            gwait(0)
            scat(0)
            gissue(0, (NCHS - 1) * CHUNK)
            gwait(1)
            scat(1)
            gwait(2)
            scat(2)
            gwait(3)
            scat(3)
            gwait(0)
            scat(0)

        plsc.subcore_barrier()

        @pl.when(c == 0)
        def _():
            pltpu.sync_copy(acc.at[zsl], outs[0].at[zsl])

        @pl.when(c == 1)
        def _():
            pltpu.sync_copy(acc.at[zsl], outs[1].at[zsl])

        if count:
            pltpu.sync_copy(cacc.at[zsl], cbuf)
            pltpu.sync_copy(cbuf, cnto.at[pl.ds(c * NP + s * RPZ, RPZ)])

    return k


_agg_l1 = _make_agg(splits=1, count=True)
_agg_l3 = _make_agg(splits=1, count=False)
_agg_l2 = _make_agg(splits=2, count=False)


_BR = 640


def _row_spec(w):
    return pl.BlockSpec((_BR, w), lambda i: (i, 0))


def _full_spec(a, b):
    return pl.BlockSpec((a, b), lambda i: (0, 0))


def _tc_l1(p0, p1, cnt0, cnt1, x, Wl1, Wr1, b1):
    def body(p0r, p1r, c0r, c1r, xr, wl, wr, br, out, cnto):
        cnt = c0r[...] + c1r[...]
        inv = 1.0 / jnp.maximum(cnt, 1.0)
        z = (jnp.dot((p0r[...] + p1r[...]) * inv, wl[...],
                     preferred_element_type=jnp.float32)
             + jnp.dot(xr[...], wr[...], preferred_element_type=jnp.float32)
             + br[...])
        out[...] = jnp.maximum(z, 0.0)
        cnto[...] = cnt

    return pl.pallas_call(
        body,
        grid=(NP // _BR,),
        in_specs=[_row_spec(128), _row_spec(128), _row_spec(1), _row_spec(1),
                  _row_spec(128), _full_spec(128, 256), _full_spec(128, 256),
                  _full_spec(1, 256)],
        out_specs=[_row_spec(256), _row_spec(1)],
        out_shape=[jax.ShapeDtypeStruct((NP, 256), jnp.float32),
                   jax.ShapeDtypeStruct((NP, 1), jnp.float32)],
    )(p0, p1, cnt0, cnt1, x, Wl1, Wr1, b1)


def _tc_l2(a0, a1, cnt, h1, Wl2a, Wl2b, Wr2, b2, Wl3):
    def body(a0r, a1r, cr, hr, wla, wlb, wr, br, wl3, h2o, y3o):
        inv = 1.0 / jnp.maximum(cr[...], 1.0)
        z = (jnp.dot(a0r[...] * inv, wla[...], preferred_element_type=jnp.float32)
             + jnp.dot(a1r[...] * inv, wlb[...], preferred_element_type=jnp.float32)
             + jnp.dot(hr[...], wr[...], preferred_element_type=jnp.float32)
             + br[...])
        h2 = jnp.maximum(z, 0.0)
        h2o[...] = h2
        y3o[...] = jnp.dot(h2, wl3[...], preferred_element_type=jnp.float32)

    return pl.pallas_call(
        body,
        grid=(NP // _BR,),
        in_specs=[_row_spec(128), _row_spec(128), _row_spec(1),
                  _row_spec(256), _full_spec(128, 256), _full_spec(128, 256),
                  _full_spec(256, 256), _full_spec(1, 256),
                  _full_spec(256, 128)],
        out_specs=[_row_spec(256), _row_spec(128)],
        out_shape=[jax.ShapeDtypeStruct((NP, 256), jnp.float32),
                   jax.ShapeDtypeStruct((NP, 128), jnp.float32)],
    )(a0, a1, cnt, h1, Wl2a, Wl2b, Wr2, b2, Wl3)


def _tc_l3(q0, q1, cnt, h2, Wr3, b3):
    def body(q0r, q1r, cr, hr, wr, br, out):
        inv = 1.0 / jnp.maximum(cr[...], 1.0)
        out[...] = ((q0r[...] + q1r[...]) * inv
                    + jnp.dot(hr[...], wr[...],
                              preferred_element_type=jnp.float32)
                    + br[...])

    return pl.pallas_call(
        body,
        grid=(NP // _BR,),
        in_specs=[_row_spec(128), _row_spec(128), _row_spec(1),
                  _row_spec(256), _full_spec(256, 128), _full_spec(1, 128)],
        out_specs=_row_spec(128),
        out_shape=jax.ShapeDtypeStruct((NP, 128), jnp.float32),
    )(q0, q1, cnt, h2, Wr3, b3)


def kernel(x, edge_index, Wl1, Wr1, b1, Wl2, Wr2, b2, Wl3, Wr3, b3):
    src = edge_index[0].astype(jnp.int32)
    dst = edge_index[1].astype(jnp.int32)
    zrows = jnp.zeros((NP, W), jnp.float32)
    zn = jnp.zeros((NP,), jnp.float32)
    ones = jnp.ones((CHUNK,), jnp.float32)

    xp = jnp.zeros((NP, 128), jnp.float32).at[:N].set(x)
    p0, p1, cnt2 = _agg_l1(xp, src, dst, zrows, zn, ones)
    cnt2 = cnt2.reshape(2, NP, 1)
    h1, cnt = _tc_l1(p0, p1, cnt2[0], cnt2[1], xp, Wl1, Wr1,
                     b1.reshape(1, 256))

    a0, a1 = _agg_l2(h1.reshape(2 * NP, W), src, dst, zrows)
    h2, y3 = _tc_l2(a0, a1, cnt, h1, Wl2[:W], Wl2[W:], Wr2,
                    b2.reshape(1, 256), Wl3)

    q0, q1 = _agg_l3(y3, src, dst, zrows)
    out = _tc_l3(q0, q1, cnt, h2, Wr3, b3.reshape(1, 128))
    return out[:N]
